# Initial kernel scaffold; baseline (speedup 1.0000x reference)
import jax
import jax.numpy as jnp
from jax.experimental import pallas as pl


def kernel(x_word, table):
    raise NotImplementedError("write your pallas kernel here")


# speedup vs baseline: 2.9672x; 2.9672x over previous
import functools

import jax
import jax.numpy as jnp
from jax import lax
from jax.experimental import pallas as pl
from jax.experimental.pallas import tpu as pltpu
from jax.experimental.pallas import tpu_sc as plsc

DIM = 128
NC = 2
NS = 16
NW = NC * NS
CHUNK = 128


@functools.lru_cache(maxsize=None)
def _make_kernel(n_chunks):
    mesh = plsc.VectorSubcoreMesh(core_axis_name="c", subcore_axis_name="s")

    @functools.partial(
        pl.kernel,
        out_type=jax.ShapeDtypeStruct((NW, n_chunks, CHUNK, DIM), jnp.float32),
        mesh=mesh,
        scratch_types=[
            pltpu.VMEM((n_chunks, CHUNK), jnp.int32),
            pltpu.VMEM((CHUNK, DIM), jnp.float32),
            pltpu.SemaphoreType.DMA,
        ],
    )
    def body(idx_hbm, table_hbm, out_hbm, idx_v, rows_v, sem):
        wid = lax.axis_index("s") * NC + lax.axis_index("c")
        pltpu.sync_copy(idx_hbm.at[wid], idx_v)

        def chunk_body(j, carry):
            pltpu.async_copy(table_hbm.at[idx_v.at[j]], rows_v, sem).wait()
            pltpu.sync_copy(rows_v, out_hbm.at[wid, j])
            return carry

        lax.fori_loop(0, n_chunks, chunk_body, 0)

    return body


def kernel(x_word, table):
    B, S = x_word.shape
    n_idx = B * S
    n_chunks = n_idx // (NW * CHUNK)
    idx = x_word.astype(jnp.int32).reshape(NW, n_chunks, CHUNK)
    out = _make_kernel(n_chunks)(idx, table)
    return out.reshape(B, S, DIM)


# speedup vs baseline: 3.3531x; 1.1301x over previous
import functools

import jax
import jax.numpy as jnp
from jax import lax
from jax.experimental import pallas as pl
from jax.experimental.pallas import tpu as pltpu
from jax.experimental.pallas import tpu_sc as plsc

DIM = 128
NC = 2
NS = 16
NW = NC * NS
CHUNK = 128
NBUF = 5


@functools.lru_cache(maxsize=None)
def _make_kernel(n_chunks):
    assert n_chunks % NBUF == 0
    mesh = plsc.VectorSubcoreMesh(core_axis_name="c", subcore_axis_name="s")

    @functools.partial(
        pl.kernel,
        out_type=jax.ShapeDtypeStruct((NW, n_chunks, CHUNK, DIM), jnp.float32),
        mesh=mesh,
        scratch_types=[
            pltpu.VMEM((n_chunks, CHUNK), jnp.int32),
            [pltpu.VMEM((CHUNK, DIM), jnp.float32) for _ in range(NBUF)],
            [pltpu.SemaphoreType.DMA for _ in range(NBUF)],
            [pltpu.SemaphoreType.DMA for _ in range(NBUF)],
        ],
    )
    def body(idx_hbm, table_hbm, out_hbm, idx_v, bufs, gsems, osems):
        wid = lax.axis_index("s") * NC + lax.axis_index("c")
        pltpu.sync_copy(idx_hbm.at[wid], idx_v)

        for b in range(NBUF):
            pltpu.async_copy(table_hbm.at[idx_v.at[b]], bufs[b], gsems[b])

        def group(g, carry):
            for b in range(NBUF):
                j = g * NBUF + b
                pltpu.make_async_copy(
                    table_hbm.at[idx_v.at[j]], bufs[b], gsems[b]).wait()
                pltpu.async_copy(bufs[b], out_hbm.at[wid, j], osems[b])
                nxt = j + NBUF

                @pl.when(nxt < n_chunks)
                def _():
                    pltpu.make_async_copy(
                        bufs[b], out_hbm.at[wid, j], osems[b]).wait()
                    pltpu.async_copy(
                        table_hbm.at[idx_v.at[nxt]], bufs[b], gsems[b])

            return carry

        lax.fori_loop(0, n_chunks // NBUF, group, 0)

        for b in range(NBUF):
            j = n_chunks - NBUF + b
            pltpu.make_async_copy(bufs[b], out_hbm.at[wid, j], osems[b]).wait()

    return body


def kernel(x_word, table):
    B, S = x_word.shape
    n_idx = B * S
    n_chunks = n_idx // (NW * CHUNK)
    idx = x_word.astype(jnp.int32).reshape(NW, n_chunks, CHUNK)
    out = _make_kernel(n_chunks)(idx, table)
    return out.reshape(B, S, DIM)


# speedup vs baseline: 5.9929x; 1.7873x over previous
import functools

import jax
import jax.numpy as jnp
from jax import lax
from jax.experimental import pallas as pl
from jax.experimental.pallas import tpu as pltpu
from jax.experimental.pallas import tpu_sc as plsc

DIM = 128
NC = 2
NS = 16
NW = NC * NS
NBUF = 8


@functools.lru_cache(maxsize=None)
def _make_kernel(B, S):
    b_per_w = B // NW
    assert b_per_w % NBUF == 0
    mesh = plsc.VectorSubcoreMesh(core_axis_name="c", subcore_axis_name="s")

    @functools.partial(
        pl.kernel,
        out_type=jax.ShapeDtypeStruct((B, S, DIM), jnp.float32),
        mesh=mesh,
        scratch_types=[
            pltpu.VMEM((b_per_w, S), jnp.int32),
            [pltpu.VMEM((S, DIM), jnp.float32) for _ in range(NBUF)],
            [pltpu.SemaphoreType.DMA for _ in range(NBUF)],
            [pltpu.SemaphoreType.DMA for _ in range(NBUF)],
        ],
    )
    def body(idx_hbm, table_hbm, out_hbm, idx_v, bufs, gsems, osems):
        wid = lax.axis_index("s") * NC + lax.axis_index("c")
        b0 = wid * b_per_w
        pltpu.sync_copy(idx_hbm.at[pl.ds(b0, b_per_w)], idx_v)

        for k in range(NBUF):
            pltpu.async_copy(table_hbm.at[idx_v.at[k]], bufs[k], gsems[k])

        def group(g, carry):
            for k in range(NBUF):
                j = g * NBUF + k
                pltpu.make_async_copy(
                    table_hbm.at[idx_v.at[j]], bufs[k], gsems[k]).wait()
                pltpu.async_copy(bufs[k], out_hbm.at[b0 + j], osems[k])
                nxt = j + NBUF

                @pl.when(nxt < b_per_w)
                def _():
                    pltpu.make_async_copy(
                        bufs[k], out_hbm.at[b0 + j], osems[k]).wait()
                    pltpu.async_copy(
                        table_hbm.at[idx_v.at[nxt]], bufs[k], gsems[k])

            return carry

        lax.fori_loop(0, b_per_w // NBUF, group, 0)

        for k in range(NBUF):
            j = b_per_w - NBUF + k
            pltpu.make_async_copy(bufs[k], out_hbm.at[b0 + j], osems[k]).wait()

    return body


def kernel(x_word, table):
    B, S = x_word.shape
    idx = x_word.astype(jnp.int32)
    return _make_kernel(B, S)(idx, table)


# speedup vs baseline: 5.9986x; 1.0009x over previous
import functools

import jax
import jax.numpy as jnp
from jax import lax
from jax.experimental import pallas as pl
from jax.experimental.pallas import tpu as pltpu
from jax.experimental.pallas import tpu_sc as plsc

DIM = 128
NC = 2
NS = 16
NW = NC * NS
NBUF = 8


@functools.lru_cache(maxsize=None)
def _make_kernel(B, S):
    b_per_w = B // NW
    assert b_per_w % NBUF == 0
    mesh = plsc.VectorSubcoreMesh(core_axis_name="c", subcore_axis_name="s")

    @functools.partial(
        pl.kernel,
        out_type=jax.ShapeDtypeStruct((B, S, DIM), jnp.float32),
        mesh=mesh,
        scratch_types=[
            pltpu.VMEM((b_per_w, S), jnp.int32),
            [pltpu.VMEM((S, DIM), jnp.float32) for _ in range(NBUF)],
            [pltpu.SemaphoreType.DMA for _ in range(NBUF)],
            [pltpu.SemaphoreType.DMA for _ in range(NBUF)],
        ],
        compiler_params=pltpu.CompilerParams(use_tc_tiling_on_sc=True),
    )
    def body(idx_hbm, table_hbm, out_hbm, idx_v, bufs, gsems, osems):
        wid = lax.axis_index("s") * NC + lax.axis_index("c")
        b0 = wid * b_per_w
        pltpu.sync_copy(idx_hbm.at[pl.ds(b0, b_per_w)], idx_v)

        for k in range(NBUF):
            pltpu.async_copy(table_hbm.at[idx_v.at[k]], bufs[k], gsems[k])

        def group(g, carry):
            for k in range(NBUF):
                j = g * NBUF + k
                pltpu.make_async_copy(
                    table_hbm.at[idx_v.at[j]], bufs[k], gsems[k]).wait()
                pltpu.async_copy(bufs[k], out_hbm.at[b0 + j], osems[k])
                nxt = j + NBUF

                @pl.when(nxt < b_per_w)
                def _():
                    pltpu.make_async_copy(
                        bufs[k], out_hbm.at[b0 + j], osems[k]).wait()
                    pltpu.async_copy(
                        table_hbm.at[idx_v.at[nxt]], bufs[k], gsems[k])

            return carry

        lax.fori_loop(0, b_per_w // NBUF, group, 0)

        for k in range(NBUF):
            j = b_per_w - NBUF + k
            pltpu.make_async_copy(bufs[k], out_hbm.at[b0 + j], osems[k]).wait()

    return body


def kernel(x_word, table):
    B, S = x_word.shape
    idx = x_word.astype(jnp.int32)
    return _make_kernel(B, S)(idx, table)


# speedup vs baseline: 10.6968x; 1.7832x over previous
import functools

import jax
import jax.numpy as jnp
from jax import lax
from jax.experimental import pallas as pl
from jax.experimental.pallas import tpu as pltpu
from jax.experimental.pallas import tpu_sc as plsc

DIM = 128
NC = 2
NS = 16
NW = NC * NS
NBUF = 5


@functools.lru_cache(maxsize=None)
def _make_kernel(B, S):
    b_per_w = B // NW
    assert S % NBUF == 0
    mesh = plsc.VectorSubcoreMesh(core_axis_name="c", subcore_axis_name="s")

    @functools.partial(
        pl.kernel,
        out_type=jax.ShapeDtypeStruct((S, B, DIM), jnp.float32),
        mesh=mesh,
        scratch_types=[
            pltpu.VMEM((S, b_per_w), jnp.int32),
            [pltpu.VMEM((b_per_w, DIM), jnp.float32) for _ in range(NBUF)],
            [pltpu.SemaphoreType.DMA for _ in range(NBUF)],
            [pltpu.SemaphoreType.DMA for _ in range(NBUF)],
        ],
    )
    def body(idx_hbm, table_hbm, out_hbm, idx_v, bufs, gsems, osems):
        wid = lax.axis_index("s") * NC + lax.axis_index("c")
        b0 = wid * b_per_w
        pltpu.sync_copy(idx_hbm.at[wid], idx_v)

        for k in range(NBUF):
            pltpu.async_copy(table_hbm.at[idx_v.at[k]], bufs[k], gsems[k])

        def group(g, carry):
            for k in range(NBUF):
                s = g * NBUF + k
                pltpu.make_async_copy(
                    table_hbm.at[idx_v.at[s]], bufs[k], gsems[k]).wait()
                pltpu.async_copy(
                    bufs[k], out_hbm.at[s, pl.ds(b0, b_per_w)], osems[k])
                nxt = s + NBUF

                @pl.when(nxt < S)
                def _():
                    pltpu.make_async_copy(
                        bufs[k], out_hbm.at[s, pl.ds(b0, b_per_w)],
                        osems[k]).wait()
                    pltpu.async_copy(
                        table_hbm.at[idx_v.at[nxt]], bufs[k], gsems[k])

            return carry

        lax.fori_loop(0, S // NBUF, group, 0)

        for k in range(NBUF):
            s = S - NBUF + k
            pltpu.make_async_copy(
                bufs[k], out_hbm.at[s, pl.ds(b0, b_per_w)], osems[k]).wait()

    return body


def kernel(x_word, table):
    B, S = x_word.shape
    b_per_w = B // NW
    idx = x_word.astype(jnp.int32).reshape(NW, b_per_w, S).transpose(0, 2, 1)
    out = _make_kernel(B, S)(idx, table)
    return out.transpose(1, 0, 2)


# speedup vs baseline: 10.7705x; 1.0069x over previous
import functools

import jax
import jax.numpy as jnp
from jax import lax
from jax.experimental import pallas as pl
from jax.experimental.pallas import tpu as pltpu
from jax.experimental.pallas import tpu_sc as plsc

DIM = 128
NC = 2
NS = 16
NW = NC * NS
NBUF = 5


@functools.lru_cache(maxsize=None)
def _make_kernel(B, S):
    b_per_w = B // NW
    assert S % NBUF == 0
    mesh = plsc.VectorSubcoreMesh(core_axis_name="c", subcore_axis_name="s")

    @functools.partial(
        pl.kernel,
        out_type=jax.ShapeDtypeStruct((S, B, DIM), jnp.float32),
        mesh=mesh,
        scratch_types=[
            pltpu.VMEM((S, 1, b_per_w), jnp.int32),
            [pltpu.VMEM((b_per_w, DIM), jnp.float32) for _ in range(NBUF)],
            [pltpu.SemaphoreType.DMA for _ in range(NBUF)],
            [pltpu.SemaphoreType.DMA for _ in range(NBUF)],
        ],
    )
    def body(idx_hbm, table_hbm, out_hbm, idx_v, bufs, gsems, osems):
        wid = lax.axis_index("s") * NC + lax.axis_index("c")
        b0 = wid * b_per_w
        pltpu.sync_copy(idx_hbm.at[:, pl.ds(wid, 1)], idx_v)

        for k in range(NBUF):
            pltpu.async_copy(table_hbm.at[idx_v.at[k, 0]], bufs[k], gsems[k])

        def group(g, carry):
            for k in range(NBUF):
                s = g * NBUF + k
                pltpu.make_async_copy(
                    table_hbm.at[idx_v.at[s, 0]], bufs[k], gsems[k]).wait()
                pltpu.async_copy(
                    bufs[k], out_hbm.at[s, pl.ds(b0, b_per_w)], osems[k])
                nxt = s + NBUF

                @pl.when(nxt < S)
                def _():
                    pltpu.make_async_copy(
                        bufs[k], out_hbm.at[s, pl.ds(b0, b_per_w)],
                        osems[k]).wait()
                    pltpu.async_copy(
                        table_hbm.at[idx_v.at[nxt, 0]], bufs[k], gsems[k])

            return carry

        lax.fori_loop(0, S // NBUF, group, 0)

        for k in range(NBUF):
            s = S - NBUF + k
            pltpu.make_async_copy(
                bufs[k], out_hbm.at[s, pl.ds(b0, b_per_w)], osems[k]).wait()

    return body


def kernel(x_word, table):
    B, S = x_word.shape
    b_per_w = B // NW
    idx = x_word.astype(jnp.int32).T.reshape(S, NW, b_per_w)
    out = _make_kernel(B, S)(idx, table)
    return out.transpose(1, 0, 2)
